# baseline (speedup 1.0000x reference)
import functools

import jax
import jax.numpy as jnp
from jax import lax
from jax.experimental import pallas as pl
from jax.experimental.pallas import tpu as pltpu
from jax.experimental.pallas import tpu_sc as plsc

_MAXLEN = 8192
_EMBED = 128
_BATCH = 4
_NC = 2
_NS = 16
_ROWS = _MAXLEN // (_NC * _NS)
_LANES = 16


def _add_rows(buf, pos_v):
    def row(r, carry):
        for c in range(_EMBED // _LANES):
            sl = pl.ds(c * _LANES, _LANES)
            buf[r, sl] = buf[r, sl] + pos_v[r, sl]
        return carry

    lax.fori_loop(0, _ROWS, row, 0)


def _sc_body(x_hbm, pos_hbm, out_hbm, pos_v, buf0, buf1,
             lsem0, lsem1, ssem0, ssem1):
    wid = lax.axis_index("s") * _NC + lax.axis_index("c")
    tsl = pl.ds(wid * _ROWS, _ROWS)

    bufs = (buf0, buf1)
    lsems = (lsem0, lsem1)
    ssems = (ssem0, ssem1)

    loads = {0: pltpu.async_copy(x_hbm.at[0, tsl], buf0, lsem0)}
    pltpu.sync_copy(pos_hbm.at[tsl], pos_v)

    stores = {}
    for b in range(_BATCH):
        i = b & 1
        loads[b].wait()
        if b + 1 < _BATCH:
            if b >= 1:
                stores[b - 1].wait()
            loads[b + 1] = pltpu.async_copy(
                x_hbm.at[b + 1, tsl], bufs[1 - i], lsems[1 - i])
        _add_rows(bufs[i], pos_v)
        stores[b] = pltpu.async_copy(bufs[i], out_hbm.at[b, tsl], ssems[i])
    stores[_BATCH - 2].wait()
    stores[_BATCH - 1].wait()


def kernel(x, pos_table):
    mesh = plsc.VectorSubcoreMesh(core_axis_name="c", subcore_axis_name="s",
                                  num_cores=_NC, num_subcores=_NS)
    run = pl.kernel(
        _sc_body,
        out_type=jax.ShapeDtypeStruct((_BATCH, _MAXLEN, _EMBED), jnp.float32),
        mesh=mesh,
        scratch_types=[
            pltpu.VMEM((_ROWS, _EMBED), jnp.float32),
            pltpu.VMEM((_ROWS, _EMBED), jnp.float32),
            pltpu.VMEM((_ROWS, _EMBED), jnp.float32),
            pltpu.SemaphoreType.DMA,
            pltpu.SemaphoreType.DMA,
            pltpu.SemaphoreType.DMA,
            pltpu.SemaphoreType.DMA,
        ],
    )
    return run(x, pos_table)
